# Initial kernel scaffold; baseline (speedup 1.0000x reference)
import jax
import jax.numpy as jnp
from jax.experimental import pallas as pl


def kernel(cat_idx_sex, cat_idx_education, cat_idx_marriage, pay_state_ids, pay_severities, num_values, W_sex, W_edu, W_mar, W_pay_state, w_sev, b_sev, W_numfeat, w_val, b_val, W_pos, cls_token, ln_gamma, ln_beta):
    raise NotImplementedError("write your pallas kernel here")


# speedup vs baseline: 11.2229x; 11.2229x over previous
import functools

import jax
import jax.numpy as jnp
from jax.experimental import pallas as pl


def _fused_kernel(idx_ref, pay_ref, sev_ref, val_ref,
                  t_sex_ref, t_edu_ref, t_mar_ref, t_pay_ref,
                  pay_pos_ref, num_base_ref, vecs_ref,
                  out_ref, *, bb):
    f32 = jnp.float32
    idx = idx_ref[...]
    pay = pay_ref[...]
    sev = sev_ref[...]
    vals = val_ref[...]

    t_sex = t_sex_ref[...]
    t_edu = t_edu_ref[...]
    t_mar = t_mar_ref[...]
    t_pay = t_pay_ref[...]
    pay_pos = pay_pos_ref[...]
    num_base = num_base_ref[...]
    vecs = vecs_ref[...]
    w_sev, w_val, g, b, cls_row = vecs[0], vecs[1], vecs[2], vecs[3], vecs[4]

    sex_t = jnp.where(idx[:, 0:1] == 0, t_sex[0], t_sex[1])
    edu_t = jnp.zeros_like(sex_t)
    for k in range(7):
        edu_t = edu_t + (idx[:, 1:2] == k).astype(f32) * t_edu[k]
    mar_t = jnp.zeros_like(sex_t)
    for k in range(4):
        mar_t = mar_t + (idx[:, 2:3] == k).astype(f32) * t_mar[k]
    cat_t = jnp.stack([sex_t, edu_t, mar_t], axis=1)

    pay3 = pay[:, :, None]
    pay_t = sev[:, :, None] * w_sev + pay_pos
    for k in range(4):
        pay_t = pay_t + (pay3 == k).astype(f32) * t_pay[k]

    num_t = vals[:, :, None] * w_val + num_base

    cls_t = jnp.broadcast_to(cls_row, (bb, 1, cls_row.shape[-1]))
    x = jnp.concatenate([cls_t, cat_t, pay_t, num_t], axis=1)

    m = jnp.mean(x, axis=-1, keepdims=True)
    xc = x - m
    v = jnp.mean(xc * xc, axis=-1, keepdims=True)
    out_ref[...] = xc * jax.lax.rsqrt(v + 1e-5) * g + b


def kernel(cat_idx_sex, cat_idx_education, cat_idx_marriage, pay_state_ids,
           pay_severities, num_values, W_sex, W_edu, W_mar, W_pay_state,
           w_sev, b_sev, W_numfeat, w_val, b_val, W_pos, cls_token,
           ln_gamma, ln_beta):
    B = num_values.shape[0]
    d = W_pos.shape[1]
    BB = 256
    grid = (B // BB,)

    idx_cat = jnp.stack([cat_idx_sex, cat_idx_education, cat_idx_marriage],
                        axis=1).astype(jnp.int32)
    pay_ids = pay_state_ids.astype(jnp.int32)
    t_sex = W_sex + W_pos[1]
    t_edu = W_edu + W_pos[2]
    t_mar = W_mar + W_pos[3]
    pay_pos = W_pos[4:10] + b_sev
    num_base = W_numfeat + W_pos[10:24] + b_val
    vecs = jnp.stack([w_sev, w_val, ln_gamma, ln_beta, cls_token[0, 0]])

    row_spec = lambda cols: pl.BlockSpec((BB, cols), lambda i: (i, 0))
    full = lambda shape: pl.BlockSpec(shape, lambda i: (0,) * len(shape))

    return pl.pallas_call(
        functools.partial(_fused_kernel, bb=BB),
        grid=grid,
        in_specs=[
            row_spec(3), row_spec(6), row_spec(6), row_spec(14),
            full((2, d)), full((7, d)), full((4, d)), full((4, d)),
            full((6, d)), full((14, d)), full((5, d)),
        ],
        out_specs=pl.BlockSpec((BB, 24, d), lambda i: (i, 0, 0)),
        out_shape=jax.ShapeDtypeStruct((B, 24, d), jnp.float32),
    )(idx_cat, pay_ids, pay_severities, num_values,
      t_sex, t_edu, t_mar, W_pay_state,
      pay_pos, num_base, vecs)
